# Initial kernel scaffold; baseline (speedup 1.0000x reference)
import jax
import jax.numpy as jnp
from jax.experimental import pallas as pl


def kernel(x, edge_index, node_rankings, W, b):
    raise NotImplementedError("write your pallas kernel here")


# speedup vs baseline: 6.6615x; 6.6615x over previous
import functools

import jax
import jax.numpy as jnp
from jax import lax
from jax.experimental import pallas as pl
from jax.experimental.pallas import tpu as pltpu
from jax.experimental.pallas import tpu_sc as plsc

N = 10000
E = 320000
D = 128
K_RANK = 1000

NC = 2
NS = 16
NW = NC * NS
CHUNK = 128
NCHUNK = 80
EPAD = NW * NCHUNK * CHUNK
NPAD = 10240
ROWS_PER_SUB = NPAD // NS


def _linear_mask_body(x_ref, nr_ref, w_ref, b_ref, o_ref):
    h = lax.dot_general(
        x_ref[...], w_ref[...],
        dimension_numbers=(((1,), (1,)), ((), ())),
        preferred_element_type=jnp.float32,
    )
    h = h + b_ref[...]
    m = (nr_ref[...] <= K_RANK).astype(jnp.float32)
    o_ref[...] = h * m


def _linear_mask(x, nr_col, W, b_row):
    return pl.pallas_call(
        _linear_mask_body,
        out_shape=jax.ShapeDtypeStruct((N, D), jnp.float32),
    )(x, nr_col, W, b_row)


def _sc_scatter_build():
    mesh = plsc.VectorSubcoreMesh(core_axis_name="c", subcore_axis_name="s")

    @functools.partial(
        pl.kernel,
        out_type=jax.ShapeDtypeStruct((NC, NPAD, D), jnp.float32),
        mesh=mesh,
        scratch_types=[
            pltpu.VMEM((CHUNK,), jnp.int32),
            pltpu.VMEM((CHUNK,), jnp.int32),
            pltpu.VMEM((CHUNK, D), jnp.float32),
            pltpu.VMEM_SHARED((NPAD, D), jnp.float32),
            pltpu.SemaphoreType.DMA,
        ],
    )
    def sc_kernel(h_hbm, rows_hbm, cols_hbm, zeros_hbm, out_hbm,
                  row_buf, col_buf, gath, acc, sem):
        c = lax.axis_index("c")
        s = lax.axis_index("s")
        wid = c * NS + s

        pltpu.sync_copy(zeros_hbm, acc.at[pl.ds(s * ROWS_PER_SUB, ROWS_PER_SUB)])
        plsc.subcore_barrier()

        @pl.loop(0, NCHUNK)
        def _(j):
            pltpu.sync_copy(rows_hbm.at[wid, j], row_buf)
            pltpu.sync_copy(cols_hbm.at[wid, j], col_buf)
            pltpu.async_copy(h_hbm.at[col_buf], gath, sem).wait()
            pltpu.sync_copy(gath, acc.at[row_buf], add=True)

        plsc.subcore_barrier()
        pltpu.sync_copy(
            acc.at[pl.ds(s * ROWS_PER_SUB, ROWS_PER_SUB)],
            out_hbm.at[c, pl.ds(s * ROWS_PER_SUB, ROWS_PER_SUB)],
        )

    return sc_kernel


_sc_scatter = _sc_scatter_build()


def _combine_body(p_ref, o_ref):
    o_ref[...] = p_ref[0] + p_ref[1]


def _combine(partial):
    blk = 2000
    return pl.pallas_call(
        _combine_body,
        grid=(N // blk,),
        in_specs=[pl.BlockSpec((NC, blk, D), lambda i: (0, i, 0))],
        out_specs=pl.BlockSpec((blk, D), lambda i: (i, 0)),
        out_shape=jax.ShapeDtypeStruct((N, D), jnp.float32),
    )(partial)


def kernel(x, edge_index, node_rankings, W, b):
    rows = edge_index[0]
    cols = edge_index[1]
    pad = EPAD - E
    rows_p = jnp.concatenate([rows, jnp.full((pad,), N, jnp.int32)])
    cols_p = jnp.concatenate([cols, jnp.zeros((pad,), jnp.int32)])
    rows_r = rows_p.reshape(NW, NCHUNK, CHUNK)
    cols_r = cols_p.reshape(NW, NCHUNK, CHUNK)

    nr_col = node_rankings[0].reshape(N, 1)
    b_row = b.reshape(1, D)
    zeros = jnp.zeros((ROWS_PER_SUB, D), jnp.float32)

    h = _linear_mask(x, nr_col, W, b_row)
    partial = _sc_scatter(h, rows_r, cols_r, zeros)
    return _combine(partial)


# speedup vs baseline: 6.9243x; 1.0395x over previous
import functools

import jax
import jax.numpy as jnp
from jax import lax
from jax.experimental import pallas as pl
from jax.experimental.pallas import tpu as pltpu
from jax.experimental.pallas import tpu_sc as plsc

N = 10000
E = 320000
D = 128
K_RANK = 1000

NC = 2
NS = 16
NW = NC * NS
CHUNK = 128
NCHUNK = 80
EPAD = NW * NCHUNK * CHUNK
NPAD = 10240
ROWS_PER_SUB = NPAD // NS


def _linear_mask_body(x_ref, nr_ref, w_ref, b_ref, o_ref):
    h = lax.dot_general(
        x_ref[...], w_ref[...],
        dimension_numbers=(((1,), (1,)), ((), ())),
        preferred_element_type=jnp.float32,
    )
    h = h + b_ref[...]
    m = (nr_ref[...] <= K_RANK).astype(jnp.float32)
    o_ref[...] = h * m


def _linear_mask(x, nr_col, W, b_row):
    return pl.pallas_call(
        _linear_mask_body,
        out_shape=jax.ShapeDtypeStruct((N, D), jnp.float32),
    )(x, nr_col, W, b_row)


NB = 2


def _sc_scatter_build():
    mesh = plsc.VectorSubcoreMesh(core_axis_name="c", subcore_axis_name="s")

    @functools.partial(
        pl.kernel,
        out_type=jax.ShapeDtypeStruct((NC, NPAD, D), jnp.float32),
        mesh=mesh,
        scratch_types=(
            [pltpu.VMEM((2, CHUNK), jnp.int32) for _ in range(NB)]
            + [pltpu.VMEM((CHUNK, D), jnp.float32) for _ in range(NB)]
            + [pltpu.VMEM_SHARED((NPAD, D), jnp.float32)]
            + [pltpu.SemaphoreType.DMA for _ in range(2 * NB)]
        ),
    )
    def sc_kernel(h_hbm, edges_hbm, zeros_hbm, out_hbm, *scratch):
        idx = scratch[:NB]
        gath = scratch[NB:2 * NB]
        acc = scratch[2 * NB]
        sem_i = scratch[2 * NB + 1:2 * NB + 1 + NB]
        sem_g = scratch[2 * NB + 1 + NB:]

        c = lax.axis_index("c")
        s = lax.axis_index("s")
        wid = c * NS + s

        for b in range(NB):
            pltpu.async_copy(edges_hbm.at[wid, b], idx[b], sem_i[b])
        pltpu.sync_copy(zeros_hbm, acc.at[pl.ds(s * ROWS_PER_SUB, ROWS_PER_SUB)])
        plsc.subcore_barrier()

        @pl.loop(0, NCHUNK, step=NB)
        def _(j0):
            for b in range(NB):
                pltpu.make_async_copy(edges_hbm.at[wid, j0 + b], idx[b],
                                      sem_i[b]).wait()
                pltpu.async_copy(h_hbm.at[idx[b].at[1]], gath[b], sem_g[b])
            for b in range(NB):
                pltpu.make_async_copy(h_hbm.at[idx[b].at[1]], gath[b],
                                      sem_g[b]).wait()
                pltpu.sync_copy(gath[b], acc.at[idx[b].at[0]], add=True)
                nxt = j0 + NB + b

                @pl.when(nxt < NCHUNK)
                def _():
                    pltpu.async_copy(edges_hbm.at[wid, nxt], idx[b], sem_i[b])

        plsc.subcore_barrier()
        pltpu.sync_copy(
            acc.at[pl.ds(s * ROWS_PER_SUB, ROWS_PER_SUB)],
            out_hbm.at[c, pl.ds(s * ROWS_PER_SUB, ROWS_PER_SUB)],
        )

    return sc_kernel


_sc_scatter = _sc_scatter_build()


def _combine_body(p_ref, o_ref):
    o_ref[...] = p_ref[0] + p_ref[1]


def _combine(partial):
    blk = 2000
    return pl.pallas_call(
        _combine_body,
        grid=(N // blk,),
        in_specs=[pl.BlockSpec((NC, blk, D), lambda i: (0, i, 0))],
        out_specs=pl.BlockSpec((blk, D), lambda i: (i, 0)),
        out_shape=jax.ShapeDtypeStruct((N, D), jnp.float32),
    )(partial)


def kernel(x, edge_index, node_rankings, W, b):
    pad = EPAD - E
    pad_vals = jnp.stack([jnp.full((pad,), N, jnp.int32),
                          jnp.zeros((pad,), jnp.int32)])
    edges_p = jnp.concatenate([edge_index, pad_vals], axis=1)
    edges_r = edges_p.reshape(2, NW, NCHUNK, CHUNK).transpose(1, 2, 0, 3)

    nr_col = node_rankings[0].reshape(N, 1)
    b_row = b.reshape(1, D)
    zeros = jnp.zeros((ROWS_PER_SUB, D), jnp.float32)

    h = _linear_mask(x, nr_col, W, b_row)
    partial = _sc_scatter(h, edges_r, zeros)
    return _combine(partial)
